# baseline (speedup 1.0000x reference)
import jax
import jax.numpy as jnp
from jax import lax
from jax.experimental import pallas as pl
from jax.experimental.pallas import tpu as pltpu
from jax.experimental.pallas import tpu_sc as plsc

V = 1000000
D = 64
DP = 128
VP = V // 2
B = 16384
K = 20
SLOTS = 32
NC, NS = 2, 16
NW = NC * NS
BPW = B // NW
CHB = 32
NCHK = BPW // CHB
CN = CHB * K
TCOLS = 7813
FULL = 7812
ITERS = 245


def _pair_body(ctabT, xtabT, tailC, tailX, ctabP, xtabP, tbuf, obuf, sem):
    wid = lax.axis_index("s") * NC + lax.axis_index("c")
    l16 = lax.iota(jnp.int32, 16)

    def do_col(tc, tabT, tabP):
        pltpu.async_copy(tabT.at[:, pl.ds(tc * 128, 128)], tbuf, sem).wait()

        def row(jl, carry):
            for s in range(8):
                dvec = (s % 4) * 16 + l16
                vvec = jnp.zeros((16,), jnp.int32) + (2 * jl + (s // 4))
                obuf[jl, pl.ds(s * 16, 16)] = plsc.load_gather(
                    tbuf, [dvec, vvec])
            return carry

        lax.fori_loop(0, 64, row, 0)
        pltpu.sync_copy(obuf, tabP.at[pl.ds(tc * 64, 64)])

    def loop(i, carry):
        tc = wid + i * NW

        @pl.when(tc < FULL)
        def _():
            do_col(tc, ctabT, ctabP)
            do_col(tc, xtabT, xtabP)

        return carry

    lax.fori_loop(0, ITERS, loop, 0)

    @pl.when(wid == 0)
    def _():
        def do_last(tail, tabP):
            pltpu.sync_copy(tail, tbuf.at[pl.ds(0, 32)])
            pltpu.sync_copy(tbuf.at[pl.ds(0, 32)],
                            tabP.at[pl.ds(FULL * 64, 32)])

        do_last(tailC, ctabP)
        do_last(tailX, xtabP)


_pair_cache = []


def _pair_kernel():
    if not _pair_cache:
        _pair_cache.append(pl.kernel(
            _pair_body,
            out_type=(
                jax.ShapeDtypeStruct((VP, DP), jnp.float32),
                jax.ShapeDtypeStruct((VP, DP), jnp.float32),
            ),
            mesh=plsc.VectorSubcoreMesh(
                core_axis_name="c", subcore_axis_name="s",
                num_cores=NC, num_subcores=NS),
            scratch_types=[
                pltpu.VMEM((D, 128), jnp.float32),
                pltpu.VMEM((D, 128), jnp.float32),
                pltpu.SemaphoreType.DMA,
            ],
            compiler_params=pltpu.CompilerParams(needs_layout_passes=False),
        ))
    return _pair_cache[0]


def _fused_body(cids, pids, nids, ctab, xtab, s_out,
                vidxr, pidxr, nidxr, vidx, pidx, nidx0, nidx1,
                vrows, prows, nrows, stage, sem):
    wid = lax.axis_index("s") * NC + lax.axis_index("c")
    lanes = lax.iota(jnp.int32, 16)

    def chunk(c, carry):
        b0 = wid * BPW + c * CHB
        r0 = b0 * K
        pltpu.sync_copy(cids.at[pl.ds(b0, CHB)], vidxr.at[pl.ds(0, CHB)])
        pltpu.sync_copy(pids.at[pl.ds(b0, CHB)], pidxr.at[pl.ds(0, CHB)])
        pltpu.sync_copy(nids.at[pl.ds(r0, CN)], nidxr.at[pl.ds(0, CN)])
        for g in range(CHB // 16):
            vidx[pl.ds(g * 16, 16)] = vidxr[pl.ds(g * 16, 16)] >> 1
            pidx[pl.ds(g * 16, 16)] = pidxr[pl.ds(g * 16, 16)] >> 1
        for g in range(CN // 16):
            w = nidxr[pl.ds(g * 16, 16)] >> 1
            if g < 32:
                nidx0[pl.ds(g * 16, 16)] = w
            else:
                nidx1[pl.ds((g - 32) * 16, 16)] = w
        c1 = pltpu.async_copy(ctab.at[vidx], vrows, sem)
        c2 = pltpu.async_copy(xtab.at[pidx], prows, sem)
        c3 = pltpu.async_copy(xtab.at[nidx0], nrows.at[pl.ds(0, 512)], sem)
        c4 = pltpu.async_copy(xtab.at[nidx1], nrows.at[pl.ds(512, 128)], sem)
        c1.wait()
        c2.wait()
        c3.wait()
        c4.wait()

        def per_b(b, carry_b):
            vo = (vidxr[pl.ds(b, 16)][0] & 1) * D
            v0 = vrows[b, pl.ds(vo, 16)]
            v1 = vrows[b, pl.ds(vo + 16, 16)]
            v2 = vrows[b, pl.ds(vo + 32, 16)]
            v3 = vrows[b, pl.ds(vo + 48, 16)]
            acc_a = jnp.zeros((16,), jnp.float32)
            acc_b = jnp.zeros((16,), jnp.float32)
            for k in range(K):
                r = b * K + k
                no = (nidxr[pl.ds(r, 16)][0] & 1) * D
                t = (v0 * nrows[r, pl.ds(no, 16)]
                     + v1 * nrows[r, pl.ds(no + 16, 16)]
                     + v2 * nrows[r, pl.ds(no + 32, 16)]
                     + v3 * nrows[r, pl.ds(no + 48, 16)])
                s = jnp.sum(t)
                if k < 16:
                    acc_a = jnp.where(lanes == k, s, acc_a)
                else:
                    acc_b = jnp.where(lanes == (k - 16), s, acc_b)
            po = (pidxr[pl.ds(b, 16)][0] & 1) * D
            t = (v0 * prows[b, pl.ds(po, 16)]
                 + v1 * prows[b, pl.ds(po + 16, 16)]
                 + v2 * prows[b, pl.ds(po + 32, 16)]
                 + v3 * prows[b, pl.ds(po + 48, 16)])
            acc_b = jnp.where(lanes == (K - 16), jnp.sum(t), acc_b)
            stage[pl.ds(b * SLOTS, 16)] = acc_a
            stage[pl.ds(b * SLOTS + 16, 16)] = acc_b
            return carry_b

        lax.fori_loop(0, CHB, per_b, 0)
        pltpu.sync_copy(stage, s_out.at[pl.ds(b0 * SLOTS, CHB * SLOTS)])
        return carry

    lax.fori_loop(0, NCHK, chunk, 0)


_fused_cache = []


def _fused_kernel():
    if not _fused_cache:
        _fused_cache.append(pl.kernel(
            _fused_body,
            out_type=jax.ShapeDtypeStruct((B * SLOTS,), jnp.float32),
            mesh=plsc.VectorSubcoreMesh(
                core_axis_name="c", subcore_axis_name="s",
                num_cores=NC, num_subcores=NS),
            scratch_types=[
                pltpu.VMEM((CHB + 16,), jnp.int32),
                pltpu.VMEM((CHB + 16,), jnp.int32),
                pltpu.VMEM((CN + 16,), jnp.int32),
                pltpu.VMEM((CHB,), jnp.int32),
                pltpu.VMEM((CHB,), jnp.int32),
                pltpu.VMEM((512,), jnp.int32),
                pltpu.VMEM((128,), jnp.int32),
                pltpu.VMEM((CHB, DP), jnp.float32),
                pltpu.VMEM((CHB, DP), jnp.float32),
                pltpu.VMEM((CN, DP), jnp.float32),
                pltpu.VMEM((CHB * SLOTS,), jnp.float32),
                pltpu.SemaphoreType.DMA,
            ],
            compiler_params=pltpu.CompilerParams(needs_layout_passes=False),
        ))
    return _fused_cache[0]


def _logsig(x):
    return jnp.minimum(x, 0.0) - jnp.log1p(jnp.exp(-jnp.abs(x)))


def _loss_body(s_ref, o_ref):
    s = s_ref[...]
    col = lax.broadcasted_iota(jnp.int32, (B, SLOTS), 1)
    neg = jnp.where(col < K, -_logsig(-s), 0.0)
    pos = jnp.where(col == K, -_logsig(s), 0.0)
    o_ref[...] = jnp.sum(neg + pos).reshape(1, 1)


_loss = pl.pallas_call(
    _loss_body,
    out_shape=jax.ShapeDtypeStruct((1, 1), jnp.float32),
)


def kernel(center_ids, pos_ids, neg_ids, center_table, context_table):
    cids = center_ids.astype(jnp.int32)
    pids = pos_ids.astype(jnp.int32)
    nids = neg_ids.reshape(-1).astype(jnp.int32)
    tailC = center_table[FULL * 128:].reshape(32, DP)
    tailX = context_table[FULL * 128:].reshape(32, DP)
    ctabP, xtabP = _pair_kernel()(center_table.T, context_table.T,
                                  tailC, tailX)
    scores = _fused_kernel()(cids, pids, nids, ctabP, xtabP)
    total = _loss(scores.reshape(B, SLOTS))
    return total[0, 0] / B


# speedup vs baseline: 2.2001x; 2.2001x over previous
import jax
import jax.numpy as jnp
from jax import lax
from jax.experimental import pallas as pl
from jax.experimental.pallas import tpu as pltpu
from jax.experimental.pallas import tpu_sc as plsc

V = 1000000
D = 64
DP = 128
VP = V // 2
B = 16384
K = 20
SLOTS = 32
NC, NS = 2, 16
NW = NC * NS
BPW = B // NW
CHB = 32
NCHK = BPW // CHB
CN = CHB * K
TCOLS = 7813
FULL = 7812
ITERS = 245


def _pair_body(ctabT, xtabT, tailC, tailX, ctabP, xtabP,
               tb0, ob0, tb1, ob1, isem0, isem1, osem0, osem1):
    wid = lax.axis_index("s") * NC + lax.axis_index("c")
    l16 = lax.iota(jnp.int32, 16)

    def run_table(tabT, tabP):
        def col(i):
            return wid + i * NW

        def start_in(tc, tb, isem):
            pltpu.async_copy(tabT.at[:, pl.ds(tc * 128, 128)], tb, isem)

        @pl.when(col(0) < FULL)
        def _():
            start_in(col(0), tb0, isem0)

        @pl.when(col(1) < FULL)
        def _():
            start_in(col(1), tb1, isem1)

        def step(i, tb, ob, isem, osem):
            tc = col(i)

            @pl.when(tc < FULL)
            def _():
                pltpu.make_async_copy(
                    tabT.at[:, pl.ds(tc * 128, 128)], tb, isem).wait()

                @pl.when(i >= 2)
                def _():
                    pltpu.make_async_copy(
                        ob, tabP.at[pl.ds(tc * 64, 64)], osem).wait()

                @plsc.parallel_loop(0, 64, unroll=8)
                def _t(jl):
                    for s in range(8):
                        dvec = (s % 4) * 16 + l16
                        vvec = jnp.zeros((16,), jnp.int32) + (2 * jl + s // 4)
                        ob[jl, pl.ds(s * 16, 16)] = plsc.load_gather(
                            tb, [dvec, vvec])

                pltpu.async_copy(ob, tabP.at[pl.ds(tc * 64, 64)], osem)
                nxt = col(i + 2)

                @pl.when(nxt < FULL)
                def _():
                    start_in(nxt, tb, isem)

        def body(j, carry):
            step(2 * j, tb0, ob0, isem0, osem0)
            step(2 * j + 1, tb1, ob1, isem1, osem1)
            return carry

        lax.fori_loop(0, (ITERS + 1) // 2, body, 0)
        pltpu.make_async_copy(ob0, tabP.at[pl.ds(0, 64)], osem0).wait()
        pltpu.make_async_copy(ob1, tabP.at[pl.ds(0, 64)], osem1).wait()

    run_table(ctabT, ctabP)
    run_table(xtabT, xtabP)

    @pl.when(wid == 0)
    def _():
        def do_last(tail, tabP):
            pltpu.sync_copy(tail, tb0.at[pl.ds(0, 32)])
            pltpu.sync_copy(tb0.at[pl.ds(0, 32)],
                            tabP.at[pl.ds(FULL * 64, 32)])

        do_last(tailC, ctabP)
        do_last(tailX, xtabP)


_pair_cache = []


def _pair_kernel():
    if not _pair_cache:
        _pair_cache.append(pl.kernel(
            _pair_body,
            out_type=(
                jax.ShapeDtypeStruct((VP, DP), jnp.float32),
                jax.ShapeDtypeStruct((VP, DP), jnp.float32),
            ),
            mesh=plsc.VectorSubcoreMesh(
                core_axis_name="c", subcore_axis_name="s",
                num_cores=NC, num_subcores=NS),
            scratch_types=[
                pltpu.VMEM((D, 128), jnp.float32),
                pltpu.VMEM((D, 128), jnp.float32),
                pltpu.VMEM((D, 128), jnp.float32),
                pltpu.VMEM((D, 128), jnp.float32),
                pltpu.SemaphoreType.DMA,
                pltpu.SemaphoreType.DMA,
                pltpu.SemaphoreType.DMA,
                pltpu.SemaphoreType.DMA,
            ],
            compiler_params=pltpu.CompilerParams(needs_layout_passes=False),
        ))
    return _pair_cache[0]


def _fused_body(cids, pids, nids, ctab, xtab, s_out,
                vidxr, pidxr, nidxr, vidx, pidx, nidx0, nidx1,
                vrows, prows, nrows, stage, sem):
    wid = lax.axis_index("s") * NC + lax.axis_index("c")
    lanes = lax.iota(jnp.int32, 16)

    def chunk(c, carry):
        b0 = wid * BPW + c * CHB
        r0 = b0 * K
        pltpu.sync_copy(cids.at[pl.ds(b0, CHB)], vidxr.at[pl.ds(0, CHB)])
        pltpu.sync_copy(pids.at[pl.ds(b0, CHB)], pidxr.at[pl.ds(0, CHB)])
        pltpu.sync_copy(nids.at[pl.ds(r0, CN)], nidxr.at[pl.ds(0, CN)])
        for g in range(CHB // 16):
            vidx[pl.ds(g * 16, 16)] = vidxr[pl.ds(g * 16, 16)] >> 1
            pidx[pl.ds(g * 16, 16)] = pidxr[pl.ds(g * 16, 16)] >> 1
        for g in range(CN // 16):
            w = nidxr[pl.ds(g * 16, 16)] >> 1
            if g < 32:
                nidx0[pl.ds(g * 16, 16)] = w
            else:
                nidx1[pl.ds((g - 32) * 16, 16)] = w
        c1 = pltpu.async_copy(ctab.at[vidx], vrows, sem)
        c2 = pltpu.async_copy(xtab.at[pidx], prows, sem)
        c3 = pltpu.async_copy(xtab.at[nidx0], nrows.at[pl.ds(0, 512)], sem)
        c4 = pltpu.async_copy(xtab.at[nidx1], nrows.at[pl.ds(512, 128)], sem)
        c1.wait()
        c2.wait()
        c3.wait()
        c4.wait()

        def per_b(b, carry_b):
            vo = (vidxr[pl.ds(b, 16)][0] & 1) * D
            v0 = vrows[b, pl.ds(vo, 16)]
            v1 = vrows[b, pl.ds(vo + 16, 16)]
            v2 = vrows[b, pl.ds(vo + 32, 16)]
            v3 = vrows[b, pl.ds(vo + 48, 16)]
            acc_a = jnp.zeros((16,), jnp.float32)
            acc_b = jnp.zeros((16,), jnp.float32)
            for k in range(K):
                r = b * K + k
                no = (nidxr[pl.ds(r, 16)][0] & 1) * D
                t = (v0 * nrows[r, pl.ds(no, 16)]
                     + v1 * nrows[r, pl.ds(no + 16, 16)]
                     + v2 * nrows[r, pl.ds(no + 32, 16)]
                     + v3 * nrows[r, pl.ds(no + 48, 16)])
                s = jnp.sum(t)
                if k < 16:
                    acc_a = jnp.where(lanes == k, s, acc_a)
                else:
                    acc_b = jnp.where(lanes == (k - 16), s, acc_b)
            po = (pidxr[pl.ds(b, 16)][0] & 1) * D
            t = (v0 * prows[b, pl.ds(po, 16)]
                 + v1 * prows[b, pl.ds(po + 16, 16)]
                 + v2 * prows[b, pl.ds(po + 32, 16)]
                 + v3 * prows[b, pl.ds(po + 48, 16)])
            acc_b = jnp.where(lanes == (K - 16), jnp.sum(t), acc_b)
            stage[pl.ds(b * SLOTS, 16)] = acc_a
            stage[pl.ds(b * SLOTS + 16, 16)] = acc_b
            return carry_b

        lax.fori_loop(0, CHB, per_b, 0)
        pltpu.sync_copy(stage, s_out.at[pl.ds(b0 * SLOTS, CHB * SLOTS)])
        return carry

    lax.fori_loop(0, NCHK, chunk, 0)


_fused_cache = []


def _fused_kernel():
    if not _fused_cache:
        _fused_cache.append(pl.kernel(
            _fused_body,
            out_type=jax.ShapeDtypeStruct((B * SLOTS,), jnp.float32),
            mesh=plsc.VectorSubcoreMesh(
                core_axis_name="c", subcore_axis_name="s",
                num_cores=NC, num_subcores=NS),
            scratch_types=[
                pltpu.VMEM((CHB + 16,), jnp.int32),
                pltpu.VMEM((CHB + 16,), jnp.int32),
                pltpu.VMEM((CN + 16,), jnp.int32),
                pltpu.VMEM((CHB,), jnp.int32),
                pltpu.VMEM((CHB,), jnp.int32),
                pltpu.VMEM((512,), jnp.int32),
                pltpu.VMEM((128,), jnp.int32),
                pltpu.VMEM((CHB, DP), jnp.float32),
                pltpu.VMEM((CHB, DP), jnp.float32),
                pltpu.VMEM((CN, DP), jnp.float32),
                pltpu.VMEM((CHB * SLOTS,), jnp.float32),
                pltpu.SemaphoreType.DMA,
            ],
            compiler_params=pltpu.CompilerParams(needs_layout_passes=False),
        ))
    return _fused_cache[0]


def _logsig(x):
    return jnp.minimum(x, 0.0) - jnp.log1p(jnp.exp(-jnp.abs(x)))


def _loss_body(s_ref, o_ref):
    s = s_ref[...]
    col = lax.broadcasted_iota(jnp.int32, (B, SLOTS), 1)
    neg = jnp.where(col < K, -_logsig(-s), 0.0)
    pos = jnp.where(col == K, -_logsig(s), 0.0)
    o_ref[...] = jnp.sum(neg + pos).reshape(1, 1)


_loss = pl.pallas_call(
    _loss_body,
    out_shape=jax.ShapeDtypeStruct((1, 1), jnp.float32),
)


def kernel(center_ids, pos_ids, neg_ids, center_table, context_table):
    cids = center_ids.astype(jnp.int32)
    pids = pos_ids.astype(jnp.int32)
    nids = neg_ids.reshape(-1).astype(jnp.int32)
    tailC = center_table[FULL * 128:].reshape(32, DP)
    tailX = context_table[FULL * 128:].reshape(32, DP)
    ctabP, xtabP = _pair_kernel()(center_table.T, context_table.T,
                                  tailC, tailX)
    scores = _fused_kernel()(cids, pids, nids, ctabP, xtabP)
    total = _loss(scores.reshape(B, SLOTS))
    return total[0, 0] / B


# speedup vs baseline: 3.2533x; 1.4788x over previous
import jax
import jax.numpy as jnp
from jax import lax
from jax.experimental import pallas as pl
from jax.experimental.pallas import tpu as pltpu
from jax.experimental.pallas import tpu_sc as plsc

V = 1000000
D = 64
DP = 128
B = 16384
K = 20
SLOTS = 32
NC, NS = 2, 16
NW = NC * NS
BPW = B // NW
CHB = 32
NCHK = BPW // CHB
CN = CHB * K


def _fused_body(cids, pids, nids, ctab, xtab, s_out,
                vidx, pidx, nidx0, nidx1,
                vrows, prows, nrows, stage, sem):
    wid = lax.axis_index("s") * NC + lax.axis_index("c")
    lanes = lax.iota(jnp.int32, 16)

    def chunk(c, carry):
        b0 = wid * BPW + c * CHB
        r0 = b0 * K
        pltpu.sync_copy(cids.at[pl.ds(b0, CHB)], vidx)
        pltpu.sync_copy(pids.at[pl.ds(b0, CHB)], pidx)
        pltpu.sync_copy(nids.at[pl.ds(r0, 512)], nidx0)
        pltpu.sync_copy(nids.at[pl.ds(r0 + 512, 128)], nidx1)
        c1 = pltpu.async_copy(ctab.at[vidx], vrows, sem)
        c2 = pltpu.async_copy(xtab.at[pidx], prows, sem)
        c3 = pltpu.async_copy(xtab.at[nidx0], nrows.at[pl.ds(0, 512)], sem)
        c4 = pltpu.async_copy(xtab.at[nidx1], nrows.at[pl.ds(512, 128)], sem)
        c1.wait()
        c2.wait()
        c3.wait()
        c4.wait()

        def per_b(b, carry_b):
            v0 = vrows[b, pl.ds(0, 16)]
            v1 = vrows[b, pl.ds(16, 16)]
            v2 = vrows[b, pl.ds(32, 16)]
            v3 = vrows[b, pl.ds(48, 16)]
            acc_a = jnp.zeros((16,), jnp.float32)
            acc_b = jnp.zeros((16,), jnp.float32)
            for k in range(K):
                r = b * K + k
                t = (v0 * nrows[r, pl.ds(0, 16)]
                     + v1 * nrows[r, pl.ds(16, 16)]
                     + v2 * nrows[r, pl.ds(32, 16)]
                     + v3 * nrows[r, pl.ds(48, 16)])
                s = jnp.sum(t)
                if k < 16:
                    acc_a = jnp.where(lanes == k, s, acc_a)
                else:
                    acc_b = jnp.where(lanes == (k - 16), s, acc_b)
            t = (v0 * prows[b, pl.ds(0, 16)] + v1 * prows[b, pl.ds(16, 16)]
                 + v2 * prows[b, pl.ds(32, 16)] + v3 * prows[b, pl.ds(48, 16)])
            acc_b = jnp.where(lanes == (K - 16), jnp.sum(t), acc_b)
            stage[pl.ds(b * SLOTS, 16)] = acc_a
            stage[pl.ds(b * SLOTS + 16, 16)] = acc_b
            return carry_b

        lax.fori_loop(0, CHB, per_b, 0)
        pltpu.sync_copy(stage, s_out.at[pl.ds(b0 * SLOTS, CHB * SLOTS)])
        return carry

    lax.fori_loop(0, NCHK, chunk, 0)


_fused_cache = []


def _fused_kernel():
    if not _fused_cache:
        _fused_cache.append(pl.kernel(
            _fused_body,
            out_type=jax.ShapeDtypeStruct((B * SLOTS,), jnp.float32),
            mesh=plsc.VectorSubcoreMesh(
                core_axis_name="c", subcore_axis_name="s",
                num_cores=NC, num_subcores=NS),
            scratch_types=[
                pltpu.VMEM((CHB,), jnp.int32),
                pltpu.VMEM((CHB,), jnp.int32),
                pltpu.VMEM((512,), jnp.int32),
                pltpu.VMEM((128,), jnp.int32),
                pltpu.VMEM((CHB, DP), jnp.float32),
                pltpu.VMEM((CHB, DP), jnp.float32),
                pltpu.VMEM((CN, DP), jnp.float32),
                pltpu.VMEM((CHB * SLOTS,), jnp.float32),
                pltpu.SemaphoreType.DMA,
            ],
            compiler_params=pltpu.CompilerParams(needs_layout_passes=False),
        ))
    return _fused_cache[0]


def _logsig(x):
    return jnp.minimum(x, 0.0) - jnp.log1p(jnp.exp(-jnp.abs(x)))


def _loss_body(s_ref, o_ref):
    s = s_ref[...]
    col = lax.broadcasted_iota(jnp.int32, (B, SLOTS), 1)
    neg = jnp.where(col < K, -_logsig(-s), 0.0)
    pos = jnp.where(col == K, -_logsig(s), 0.0)
    o_ref[...] = jnp.sum(neg + pos).reshape(1, 1)


_loss = pl.pallas_call(
    _loss_body,
    out_shape=jax.ShapeDtypeStruct((1, 1), jnp.float32),
)


def kernel(center_ids, pos_ids, neg_ids, center_table, context_table):
    cids = center_ids.astype(jnp.int32)
    pids = pos_ids.astype(jnp.int32)
    nids = neg_ids.reshape(-1).astype(jnp.int32)
    z = jnp.zeros((V, DP - D), jnp.float32)
    ctab = jnp.concatenate([center_table, z], axis=1)
    xtab = jnp.concatenate([context_table, z], axis=1)
    scores = _fused_kernel()(cids, pids, nids, ctab, xtab)
    total = _loss(scores.reshape(B, SLOTS))
    return total[0, 0] / B


# speedup vs baseline: 3.6405x; 1.1190x over previous
import jax
import jax.numpy as jnp
from jax import lax
from jax.experimental import pallas as pl
from jax.experimental.pallas import tpu as pltpu
from jax.experimental.pallas import tpu_sc as plsc

V = 1000000
D = 64
DP = 128
VP = V // 2
B = 16384
K = 20
SLOTS = 32
NC, NS = 2, 16
NW = NC * NS
BPW = B // NW
CHB = 32
NCHK = BPW // CHB
CN = CHB * K
TCOLS = 7813
FULL = 7812
ITERS = 245


def _pair_body(ctabT, xtabT, tailC, tailX, ctabP, xtabP,
               tb0, ob0, tb1, ob1, isem0, isem1, osem0, osem1):
    wid = lax.axis_index("s") * NC + lax.axis_index("c")
    l16 = lax.iota(jnp.int32, 16)

    def run_table(tabT, tabP):
        def col(i):
            return wid + i * NW

        def start_in(tc, tb, isem):
            pltpu.async_copy(tabT.at[:, pl.ds(tc * 128, 128)], tb, isem)

        @pl.when(col(0) < FULL)
        def _():
            start_in(col(0), tb0, isem0)

        @pl.when(col(1) < FULL)
        def _():
            start_in(col(1), tb1, isem1)

        def step(i, tb, ob, isem, osem):
            tc = col(i)

            @pl.when(tc < FULL)
            def _():
                pltpu.make_async_copy(
                    tabT.at[:, pl.ds(tc * 128, 128)], tb, isem).wait()

                @pl.when(i >= 2)
                def _():
                    pltpu.make_async_copy(
                        ob, tabP.at[pl.ds(tc * 64, 64)], osem).wait()

                @plsc.parallel_loop(0, 16, unroll=4)
                def _t(t):
                    perm = (l16 + t) & 15
                    for a in range(4):
                        dvec = a * 16 + l16
                        for m in range(8):
                            vv = m * 16 + perm
                            w = plsc.load_gather(tb, [dvec, vv])
                            jv = vv >> 1
                            cv = ((vv & 1) << 6) | dvec
                            plsc.store_scatter(ob, [jv, cv], w)

                pltpu.async_copy(ob, tabP.at[pl.ds(tc * 64, 64)], osem)
                nxt = col(i + 2)

                @pl.when(nxt < FULL)
                def _():
                    start_in(nxt, tb, isem)

        def body(j, carry):
            step(2 * j, tb0, ob0, isem0, osem0)
            step(2 * j + 1, tb1, ob1, isem1, osem1)
            return carry

        lax.fori_loop(0, (ITERS + 1) // 2, body, 0)
        pltpu.make_async_copy(ob0, tabP.at[pl.ds(0, 64)], osem0).wait()
        pltpu.make_async_copy(ob1, tabP.at[pl.ds(0, 64)], osem1).wait()

    run_table(ctabT, ctabP)
    run_table(xtabT, xtabP)

    @pl.when(wid == 0)
    def _():
        def do_last(tail, tabP):
            pltpu.sync_copy(tail, tb0.at[pl.ds(0, 32)])
            pltpu.sync_copy(tb0.at[pl.ds(0, 32)],
                            tabP.at[pl.ds(FULL * 64, 32)])

        do_last(tailC, ctabP)
        do_last(tailX, xtabP)


_pair_cache = []


def _pair_kernel():
    if not _pair_cache:
        _pair_cache.append(pl.kernel(
            _pair_body,
            out_type=(
                jax.ShapeDtypeStruct((VP, DP), jnp.float32),
                jax.ShapeDtypeStruct((VP, DP), jnp.float32),
            ),
            mesh=plsc.VectorSubcoreMesh(
                core_axis_name="c", subcore_axis_name="s",
                num_cores=NC, num_subcores=NS),
            scratch_types=[
                pltpu.VMEM((D, 128), jnp.float32),
                pltpu.VMEM((D, 128), jnp.float32),
                pltpu.VMEM((D, 128), jnp.float32),
                pltpu.VMEM((D, 128), jnp.float32),
                pltpu.SemaphoreType.DMA,
                pltpu.SemaphoreType.DMA,
                pltpu.SemaphoreType.DMA,
                pltpu.SemaphoreType.DMA,
            ],
            compiler_params=pltpu.CompilerParams(needs_layout_passes=False),
        ))
    return _pair_cache[0]


def _fused_body(cids, pids, nids, ctab, xtab, s_out,
                vidxr, pidxr, nidxr, vidx, pidx, nidx0, nidx1,
                vrows, prows, nrows, stage, sem):
    wid = lax.axis_index("s") * NC + lax.axis_index("c")
    lanes = lax.iota(jnp.int32, 16)

    def chunk(c, carry):
        b0 = wid * BPW + c * CHB
        r0 = b0 * K
        pltpu.sync_copy(cids.at[pl.ds(b0, CHB)], vidxr.at[pl.ds(0, CHB)])
        pltpu.sync_copy(pids.at[pl.ds(b0, CHB)], pidxr.at[pl.ds(0, CHB)])
        pltpu.sync_copy(nids.at[pl.ds(r0, CN)], nidxr.at[pl.ds(0, CN)])
        for g in range(CHB // 16):
            vidx[pl.ds(g * 16, 16)] = vidxr[pl.ds(g * 16, 16)] >> 1
            pidx[pl.ds(g * 16, 16)] = pidxr[pl.ds(g * 16, 16)] >> 1
        for g in range(CN // 16):
            w = nidxr[pl.ds(g * 16, 16)] >> 1
            if g < 32:
                nidx0[pl.ds(g * 16, 16)] = w
            else:
                nidx1[pl.ds((g - 32) * 16, 16)] = w
        c1 = pltpu.async_copy(ctab.at[vidx], vrows, sem)
        c2 = pltpu.async_copy(xtab.at[pidx], prows, sem)
        c3 = pltpu.async_copy(xtab.at[nidx0], nrows.at[pl.ds(0, 512)], sem)
        c4 = pltpu.async_copy(xtab.at[nidx1], nrows.at[pl.ds(512, 128)], sem)
        c1.wait()
        c2.wait()
        c3.wait()
        c4.wait()

        def per_b(b, carry_b):
            vo = (vidxr[pl.ds(b, 16)][0] & 1) * D
            v0 = vrows[b, pl.ds(vo, 16)]
            v1 = vrows[b, pl.ds(vo + 16, 16)]
            v2 = vrows[b, pl.ds(vo + 32, 16)]
            v3 = vrows[b, pl.ds(vo + 48, 16)]
            acc_a = jnp.zeros((16,), jnp.float32)
            acc_b = jnp.zeros((16,), jnp.float32)
            for k in range(K):
                r = b * K + k
                no = (nidxr[pl.ds(r, 16)][0] & 1) * D
                t = (v0 * nrows[r, pl.ds(no, 16)]
                     + v1 * nrows[r, pl.ds(no + 16, 16)]
                     + v2 * nrows[r, pl.ds(no + 32, 16)]
                     + v3 * nrows[r, pl.ds(no + 48, 16)])
                s = jnp.sum(t)
                if k < 16:
                    acc_a = jnp.where(lanes == k, s, acc_a)
                else:
                    acc_b = jnp.where(lanes == (k - 16), s, acc_b)
            po = (pidxr[pl.ds(b, 16)][0] & 1) * D
            t = (v0 * prows[b, pl.ds(po, 16)]
                 + v1 * prows[b, pl.ds(po + 16, 16)]
                 + v2 * prows[b, pl.ds(po + 32, 16)]
                 + v3 * prows[b, pl.ds(po + 48, 16)])
            acc_b = jnp.where(lanes == (K - 16), jnp.sum(t), acc_b)
            stage[pl.ds(b * SLOTS, 16)] = acc_a
            stage[pl.ds(b * SLOTS + 16, 16)] = acc_b
            return carry_b

        lax.fori_loop(0, CHB, per_b, 0)
        pltpu.sync_copy(stage, s_out.at[pl.ds(b0 * SLOTS, CHB * SLOTS)])
        return carry

    lax.fori_loop(0, NCHK, chunk, 0)


_fused_cache = []


def _fused_kernel():
    if not _fused_cache:
        _fused_cache.append(pl.kernel(
            _fused_body,
            out_type=jax.ShapeDtypeStruct((B * SLOTS,), jnp.float32),
            mesh=plsc.VectorSubcoreMesh(
                core_axis_name="c", subcore_axis_name="s",
                num_cores=NC, num_subcores=NS),
            scratch_types=[
                pltpu.VMEM((CHB + 16,), jnp.int32),
                pltpu.VMEM((CHB + 16,), jnp.int32),
                pltpu.VMEM((CN + 16,), jnp.int32),
                pltpu.VMEM((CHB,), jnp.int32),
                pltpu.VMEM((CHB,), jnp.int32),
                pltpu.VMEM((512,), jnp.int32),
                pltpu.VMEM((128,), jnp.int32),
                pltpu.VMEM((CHB, DP), jnp.float32),
                pltpu.VMEM((CHB, DP), jnp.float32),
                pltpu.VMEM((CN, DP), jnp.float32),
                pltpu.VMEM((CHB * SLOTS,), jnp.float32),
                pltpu.SemaphoreType.DMA,
            ],
            compiler_params=pltpu.CompilerParams(needs_layout_passes=False),
        ))
    return _fused_cache[0]


def _logsig(x):
    return jnp.minimum(x, 0.0) - jnp.log1p(jnp.exp(-jnp.abs(x)))


def _loss_body(s_ref, o_ref):
    s = s_ref[...]
    col = lax.broadcasted_iota(jnp.int32, (B, SLOTS), 1)
    neg = jnp.where(col < K, -_logsig(-s), 0.0)
    pos = jnp.where(col == K, -_logsig(s), 0.0)
    o_ref[...] = jnp.sum(neg + pos).reshape(1, 1)


_loss = pl.pallas_call(
    _loss_body,
    out_shape=jax.ShapeDtypeStruct((1, 1), jnp.float32),
)


def kernel(center_ids, pos_ids, neg_ids, center_table, context_table):
    cids = center_ids.astype(jnp.int32)
    pids = pos_ids.astype(jnp.int32)
    nids = neg_ids.reshape(-1).astype(jnp.int32)
    tailC = center_table[FULL * 128:].reshape(32, DP)
    tailX = context_table[FULL * 128:].reshape(32, DP)
    ctabP, xtabP = _pair_kernel()(center_table.T, context_table.T,
                                  tailC, tailX)
    scores = _fused_kernel()(cids, pids, nids, ctabP, xtabP)
    total = _loss(scores.reshape(B, SLOTS))
    return total[0, 0] / B


# speedup vs baseline: 3.7583x; 1.0323x over previous
import jax
import jax.numpy as jnp
from jax import lax
from jax.experimental import pallas as pl
from jax.experimental.pallas import tpu as pltpu
from jax.experimental.pallas import tpu_sc as plsc

V = 1000000
D = 64
DP = 128
VP = V // 2
B = 16384
K = 20
SLOTS = 32
NC, NS = 2, 16
NW = NC * NS
BPW = B // NW
CHB = 32
NCHK = BPW // CHB
CN = CHB * K
TCOLS = 7813
FULL = 7812
ITERS = 245


def _pair_body(ctabT, xtabT, tailC, tailX, ctabP, xtabP,
               tb0, ob0, tb1, ob1, isem0, isem1, osem0, osem1):
    wid = lax.axis_index("s") * NC + lax.axis_index("c")
    l16 = lax.iota(jnp.int32, 16)

    def run_table(tabT, tabP):
        def col(i):
            return wid + i * NW

        def start_in(tc, tb, isem):
            pltpu.async_copy(tabT.at[:, pl.ds(tc * 128, 128)], tb, isem)

        @pl.when(col(0) < FULL)
        def _():
            start_in(col(0), tb0, isem0)

        @pl.when(col(1) < FULL)
        def _():
            start_in(col(1), tb1, isem1)

        def step(i, tb, ob, isem, osem):
            tc = col(i)

            @pl.when(tc < FULL)
            def _():
                pltpu.make_async_copy(
                    tabT.at[:, pl.ds(tc * 128, 128)], tb, isem).wait()

                @pl.when(i >= 2)
                def _():
                    pltpu.make_async_copy(
                        ob, tabP.at[pl.ds(tc * 64, 64)], osem).wait()

                @plsc.parallel_loop(0, 16, unroll=4)
                def _t(t):
                    perm = (l16 + t) & 15
                    jvh = perm >> 1
                    pv6 = (perm & 1) << 6
                    for a in range(4):
                        dvec = a * 16 + l16
                        cv = pv6 | dvec
                        for m in range(8):
                            w = plsc.load_gather(tb, [dvec, m * 16 + perm])
                            plsc.store_scatter(ob, [m * 8 + jvh, cv], w)

                pltpu.async_copy(ob, tabP.at[pl.ds(tc * 64, 64)], osem)
                nxt = col(i + 2)

                @pl.when(nxt < FULL)
                def _():
                    start_in(nxt, tb, isem)

        def body(j, carry):
            step(2 * j, tb0, ob0, isem0, osem0)
            step(2 * j + 1, tb1, ob1, isem1, osem1)
            return carry

        lax.fori_loop(0, (ITERS + 1) // 2, body, 0)
        pltpu.make_async_copy(ob0, tabP.at[pl.ds(0, 64)], osem0).wait()
        pltpu.make_async_copy(ob1, tabP.at[pl.ds(0, 64)], osem1).wait()

    run_table(ctabT, ctabP)
    run_table(xtabT, xtabP)

    @pl.when(wid == 0)
    def _():
        def do_last(tail, tabP):
            pltpu.sync_copy(tail, tb0.at[pl.ds(0, 32)])
            pltpu.sync_copy(tb0.at[pl.ds(0, 32)],
                            tabP.at[pl.ds(FULL * 64, 32)])

        do_last(tailC, ctabP)
        do_last(tailX, xtabP)


_pair_cache = []


def _pair_kernel():
    if not _pair_cache:
        _pair_cache.append(pl.kernel(
            _pair_body,
            out_type=(
                jax.ShapeDtypeStruct((VP, DP), jnp.float32),
                jax.ShapeDtypeStruct((VP, DP), jnp.float32),
            ),
            mesh=plsc.VectorSubcoreMesh(
                core_axis_name="c", subcore_axis_name="s",
                num_cores=NC, num_subcores=NS),
            scratch_types=[
                pltpu.VMEM((D, 128), jnp.float32),
                pltpu.VMEM((D, 128), jnp.float32),
                pltpu.VMEM((D, 128), jnp.float32),
                pltpu.VMEM((D, 128), jnp.float32),
                pltpu.SemaphoreType.DMA,
                pltpu.SemaphoreType.DMA,
                pltpu.SemaphoreType.DMA,
                pltpu.SemaphoreType.DMA,
            ],
            compiler_params=pltpu.CompilerParams(needs_layout_passes=False),
        ))
    return _pair_cache[0]


def _fused_body(cids, pids, nids, ctab, xtab, s_out,
                vidxr, pidxr, nidxr, vidx, pidx, nidx0, nidx1,
                vrows, prows, nrows, stage, sem):
    wid = lax.axis_index("s") * NC + lax.axis_index("c")
    lanes = lax.iota(jnp.int32, 16)

    def chunk(c, carry):
        b0 = wid * BPW + c * CHB
        r0 = b0 * K
        pltpu.sync_copy(cids.at[pl.ds(b0, CHB)], vidxr.at[pl.ds(0, CHB)])
        pltpu.sync_copy(pids.at[pl.ds(b0, CHB)], pidxr.at[pl.ds(0, CHB)])
        pltpu.sync_copy(nids.at[pl.ds(r0, CN)], nidxr.at[pl.ds(0, CN)])
        for g in range(CHB // 16):
            vidx[pl.ds(g * 16, 16)] = vidxr[pl.ds(g * 16, 16)] >> 1
            pidx[pl.ds(g * 16, 16)] = pidxr[pl.ds(g * 16, 16)] >> 1
        for g in range(CN // 16):
            w = nidxr[pl.ds(g * 16, 16)] >> 1
            if g < 32:
                nidx0[pl.ds(g * 16, 16)] = w
            else:
                nidx1[pl.ds((g - 32) * 16, 16)] = w
        c1 = pltpu.async_copy(ctab.at[vidx], vrows, sem)
        c2 = pltpu.async_copy(xtab.at[pidx], prows, sem)
        c3 = pltpu.async_copy(xtab.at[nidx0], nrows.at[pl.ds(0, 512)], sem)
        c4 = pltpu.async_copy(xtab.at[nidx1], nrows.at[pl.ds(512, 128)], sem)
        c1.wait()
        c2.wait()
        c3.wait()
        c4.wait()

        def per_b(b, carry_b):
            vo = (vidxr[pl.ds(b, 16)][0] & 1) * D
            v0 = vrows[b, pl.ds(vo, 16)]
            v1 = vrows[b, pl.ds(vo + 16, 16)]
            v2 = vrows[b, pl.ds(vo + 32, 16)]
            v3 = vrows[b, pl.ds(vo + 48, 16)]
            acc_a = jnp.zeros((16,), jnp.float32)
            acc_b = jnp.zeros((16,), jnp.float32)
            for k in range(K):
                r = b * K + k
                no = (nidxr[pl.ds(r, 16)][0] & 1) * D
                t = (v0 * nrows[r, pl.ds(no, 16)]
                     + v1 * nrows[r, pl.ds(no + 16, 16)]
                     + v2 * nrows[r, pl.ds(no + 32, 16)]
                     + v3 * nrows[r, pl.ds(no + 48, 16)])
                s = jnp.sum(t)
                if k < 16:
                    acc_a = jnp.where(lanes == k, s, acc_a)
                else:
                    acc_b = jnp.where(lanes == (k - 16), s, acc_b)
            po = (pidxr[pl.ds(b, 16)][0] & 1) * D
            t = (v0 * prows[b, pl.ds(po, 16)]
                 + v1 * prows[b, pl.ds(po + 16, 16)]
                 + v2 * prows[b, pl.ds(po + 32, 16)]
                 + v3 * prows[b, pl.ds(po + 48, 16)])
            acc_b = jnp.where(lanes == (K - 16), jnp.sum(t), acc_b)
            stage[pl.ds(b * SLOTS, 16)] = acc_a
            stage[pl.ds(b * SLOTS + 16, 16)] = acc_b
            return carry_b

        lax.fori_loop(0, CHB, per_b, 0)
        pltpu.sync_copy(stage, s_out.at[pl.ds(b0 * SLOTS, CHB * SLOTS)])
        return carry

    lax.fori_loop(0, NCHK, chunk, 0)


_fused_cache = []


def _fused_kernel():
    if not _fused_cache:
        _fused_cache.append(pl.kernel(
            _fused_body,
            out_type=jax.ShapeDtypeStruct((B * SLOTS,), jnp.float32),
            mesh=plsc.VectorSubcoreMesh(
                core_axis_name="c", subcore_axis_name="s",
                num_cores=NC, num_subcores=NS),
            scratch_types=[
                pltpu.VMEM((CHB + 16,), jnp.int32),
                pltpu.VMEM((CHB + 16,), jnp.int32),
                pltpu.VMEM((CN + 16,), jnp.int32),
                pltpu.VMEM((CHB,), jnp.int32),
                pltpu.VMEM((CHB,), jnp.int32),
                pltpu.VMEM((512,), jnp.int32),
                pltpu.VMEM((128,), jnp.int32),
                pltpu.VMEM((CHB, DP), jnp.float32),
                pltpu.VMEM((CHB, DP), jnp.float32),
                pltpu.VMEM((CN, DP), jnp.float32),
                pltpu.VMEM((CHB * SLOTS,), jnp.float32),
                pltpu.SemaphoreType.DMA,
            ],
            compiler_params=pltpu.CompilerParams(needs_layout_passes=False),
        ))
    return _fused_cache[0]


def _logsig(x):
    return jnp.minimum(x, 0.0) - jnp.log1p(jnp.exp(-jnp.abs(x)))


def _loss_body(s_ref, o_ref):
    s = s_ref[...]
    col = lax.broadcasted_iota(jnp.int32, (B, SLOTS), 1)
    neg = jnp.where(col < K, -_logsig(-s), 0.0)
    pos = jnp.where(col == K, -_logsig(s), 0.0)
    o_ref[...] = jnp.sum(neg + pos).reshape(1, 1)


_loss = pl.pallas_call(
    _loss_body,
    out_shape=jax.ShapeDtypeStruct((1, 1), jnp.float32),
)


def kernel(center_ids, pos_ids, neg_ids, center_table, context_table):
    cids = center_ids.astype(jnp.int32)
    pids = pos_ids.astype(jnp.int32)
    nids = neg_ids.reshape(-1).astype(jnp.int32)
    tailC = center_table[FULL * 128:].reshape(32, DP)
    tailX = context_table[FULL * 128:].reshape(32, DP)
    ctabP, xtabP = _pair_kernel()(center_table.T, context_table.T,
                                  tailC, tailX)
    scores = _fused_kernel()(cids, pids, nids, ctabP, xtabP)
    total = _loss(scores.reshape(B, SLOTS))
    return total[0, 0] / B


# speedup vs baseline: 6.6947x; 1.7813x over previous
import jax
import jax.numpy as jnp
from jax import lax
from jax.experimental import pallas as pl
from jax.experimental.pallas import tpu as pltpu
from jax.experimental.pallas import tpu_sc as plsc

V = 1000000
D = 64
DP = 128
VP = V // 2
B = 16384
K = 20
SLOTS = 32
NC, NS = 2, 16
NW = NC * NS
BPW = B // NW
CHB = 32
NCHK = BPW // CHB
CN = CHB * K
TCOLS = 7813
FULL = 7812
QCOLS = FULL // 2
ITERS = 123


def _pair_body(ctabT, xtabT, tailC, tailX, ctabP, xtabP,
               tb0, ob0, tb1, ob1, isem0, isem1, osem0, osem1):
    wid = lax.axis_index("s") * NC + lax.axis_index("c")
    l16 = lax.iota(jnp.int32, 16)

    def run_table(tabT, tabP):
        def col(i):
            return wid + i * NW

        def start_in(q, tb, isem):
            pltpu.async_copy(tabT.at[:, pl.ds(q * 256, 256)], tb, isem)

        @pl.when(col(0) < QCOLS)
        def _():
            start_in(col(0), tb0, isem0)

        @pl.when(col(1) < QCOLS)
        def _():
            start_in(col(1), tb1, isem1)

        def step(i, tb, ob, isem, osem):
            q = col(i)

            @pl.when(q < QCOLS)
            def _():
                pltpu.make_async_copy(
                    tabT.at[:, pl.ds(q * 256, 256)], tb, isem).wait()

                @pl.when(i >= 2)
                def _():
                    pltpu.make_async_copy(
                        ob, tabP.at[pl.ds(q * 128, 128)], osem).wait()

                @plsc.parallel_loop(0, 16, unroll=4)
                def _t(t):
                    perm = (l16 + t) & 15
                    jvh = perm >> 1
                    pv6 = (perm & 1) << 6
                    for a in range(4):
                        dvec = a * 16 + l16
                        cv = pv6 | dvec
                        for m in range(16):
                            w = plsc.load_gather(tb, [dvec, m * 16 + perm])
                            plsc.store_scatter(ob, [m * 8 + jvh, cv], w)

                pltpu.async_copy(ob, tabP.at[pl.ds(q * 128, 128)], osem)
                nxt = col(i + 2)

                @pl.when(nxt < QCOLS)
                def _():
                    start_in(nxt, tb, isem)

        def body(j, carry):
            step(2 * j, tb0, ob0, isem0, osem0)
            step(2 * j + 1, tb1, ob1, isem1, osem1)
            return carry

        lax.fori_loop(0, (ITERS + 1) // 2, body, 0)
        pltpu.make_async_copy(ob0, tabP.at[pl.ds(0, 128)], osem0).wait()
        pltpu.make_async_copy(ob1, tabP.at[pl.ds(0, 128)], osem1).wait()

    run_table(ctabT, ctabP)
    run_table(xtabT, xtabP)

    @pl.when(wid == 0)
    def _():
        def do_last(tail, tabP):
            pltpu.sync_copy(tail, ob0.at[pl.ds(0, 32)])
            pltpu.sync_copy(ob0.at[pl.ds(0, 32)],
                            tabP.at[pl.ds(FULL * 64, 32)])

        do_last(tailC, ctabP)
        do_last(tailX, xtabP)


_pair_cache = []


def _pair_kernel():
    if not _pair_cache:
        _pair_cache.append(pl.kernel(
            _pair_body,
            out_type=(
                jax.ShapeDtypeStruct((VP, DP), jnp.float32),
                jax.ShapeDtypeStruct((VP, DP), jnp.float32),
            ),
            mesh=plsc.VectorSubcoreMesh(
                core_axis_name="c", subcore_axis_name="s",
                num_cores=NC, num_subcores=NS),
            scratch_types=[
                pltpu.VMEM((D, 256), jnp.float32),
                pltpu.VMEM((128, 128), jnp.float32),
                pltpu.VMEM((D, 256), jnp.float32),
                pltpu.VMEM((128, 128), jnp.float32),
                pltpu.SemaphoreType.DMA,
                pltpu.SemaphoreType.DMA,
                pltpu.SemaphoreType.DMA,
                pltpu.SemaphoreType.DMA,
            ],
            compiler_params=pltpu.CompilerParams(needs_layout_passes=False),
        ))
    return _pair_cache[0]


def _fused_body(cids, pids, nids, ctab, xtab, s_out,
                vidxr, pidxr, nidxr, vidx, pidx, nidx0, nidx1,
                vrows, prows, nrows, stage, sem):
    wid = lax.axis_index("s") * NC + lax.axis_index("c")
    lanes = lax.iota(jnp.int32, 16)

    def chunk(c, carry):
        b0 = wid * BPW + c * CHB
        r0 = b0 * K
        pltpu.sync_copy(cids.at[pl.ds(b0, CHB)], vidxr.at[pl.ds(0, CHB)])
        pltpu.sync_copy(pids.at[pl.ds(b0, CHB)], pidxr.at[pl.ds(0, CHB)])
        pltpu.sync_copy(nids.at[pl.ds(r0, CN)], nidxr.at[pl.ds(0, CN)])
        for g in range(CHB // 16):
            vidx[pl.ds(g * 16, 16)] = vidxr[pl.ds(g * 16, 16)] >> 1
            pidx[pl.ds(g * 16, 16)] = pidxr[pl.ds(g * 16, 16)] >> 1
        for g in range(CN // 16):
            w = nidxr[pl.ds(g * 16, 16)] >> 1
            if g < 32:
                nidx0[pl.ds(g * 16, 16)] = w
            else:
                nidx1[pl.ds((g - 32) * 16, 16)] = w
        c1 = pltpu.async_copy(ctab.at[vidx], vrows, sem)
        c2 = pltpu.async_copy(xtab.at[pidx], prows, sem)
        c3 = pltpu.async_copy(xtab.at[nidx0], nrows.at[pl.ds(0, 512)], sem)
        c4 = pltpu.async_copy(xtab.at[nidx1], nrows.at[pl.ds(512, 128)], sem)
        c1.wait()
        c2.wait()
        c3.wait()
        c4.wait()

        def per_b(b, carry_b):
            vo = (vidxr[pl.ds(b, 16)][0] & 1) * D
            v0 = vrows[b, pl.ds(vo, 16)]
            v1 = vrows[b, pl.ds(vo + 16, 16)]
            v2 = vrows[b, pl.ds(vo + 32, 16)]
            v3 = vrows[b, pl.ds(vo + 48, 16)]
            acc_a = jnp.zeros((16,), jnp.float32)
            acc_b = jnp.zeros((16,), jnp.float32)
            for k in range(K):
                r = b * K + k
                no = (nidxr[pl.ds(r, 16)][0] & 1) * D
                t = (v0 * nrows[r, pl.ds(no, 16)]
                     + v1 * nrows[r, pl.ds(no + 16, 16)]
                     + v2 * nrows[r, pl.ds(no + 32, 16)]
                     + v3 * nrows[r, pl.ds(no + 48, 16)])
                s = jnp.sum(t)
                if k < 16:
                    acc_a = jnp.where(lanes == k, s, acc_a)
                else:
                    acc_b = jnp.where(lanes == (k - 16), s, acc_b)
            po = (pidxr[pl.ds(b, 16)][0] & 1) * D
            t = (v0 * prows[b, pl.ds(po, 16)]
                 + v1 * prows[b, pl.ds(po + 16, 16)]
                 + v2 * prows[b, pl.ds(po + 32, 16)]
                 + v3 * prows[b, pl.ds(po + 48, 16)])
            acc_b = jnp.where(lanes == (K - 16), jnp.sum(t), acc_b)
            stage[pl.ds(b * SLOTS, 16)] = acc_a
            stage[pl.ds(b * SLOTS + 16, 16)] = acc_b
            return carry_b

        lax.fori_loop(0, CHB, per_b, 0)
        pltpu.sync_copy(stage, s_out.at[pl.ds(b0 * SLOTS, CHB * SLOTS)])
        return carry

    lax.fori_loop(0, NCHK, chunk, 0)


_fused_cache = []


def _fused_kernel():
    if not _fused_cache:
        _fused_cache.append(pl.kernel(
            _fused_body,
            out_type=jax.ShapeDtypeStruct((B * SLOTS,), jnp.float32),
            mesh=plsc.VectorSubcoreMesh(
                core_axis_name="c", subcore_axis_name="s",
                num_cores=NC, num_subcores=NS),
            scratch_types=[
                pltpu.VMEM((CHB + 16,), jnp.int32),
                pltpu.VMEM((CHB + 16,), jnp.int32),
                pltpu.VMEM((CN + 16,), jnp.int32),
                pltpu.VMEM((CHB,), jnp.int32),
                pltpu.VMEM((CHB,), jnp.int32),
                pltpu.VMEM((512,), jnp.int32),
                pltpu.VMEM((128,), jnp.int32),
                pltpu.VMEM((CHB, DP), jnp.float32),
                pltpu.VMEM((CHB, DP), jnp.float32),
                pltpu.VMEM((CN, DP), jnp.float32),
                pltpu.VMEM((CHB * SLOTS,), jnp.float32),
                pltpu.SemaphoreType.DMA,
            ],
            compiler_params=pltpu.CompilerParams(needs_layout_passes=False),
        ))
    return _fused_cache[0]


def _logsig(x):
    return jnp.minimum(x, 0.0) - jnp.log1p(jnp.exp(-jnp.abs(x)))


def _loss_body(s_ref, o_ref):
    s = s_ref[...]
    col = lax.broadcasted_iota(jnp.int32, (B, SLOTS), 1)
    neg = jnp.where(col < K, -_logsig(-s), 0.0)
    pos = jnp.where(col == K, -_logsig(s), 0.0)
    o_ref[...] = jnp.sum(neg + pos).reshape(1, 1)


_loss = pl.pallas_call(
    _loss_body,
    out_shape=jax.ShapeDtypeStruct((1, 1), jnp.float32),
)


def kernel(center_ids, pos_ids, neg_ids, center_table, context_table):
    cids = center_ids.astype(jnp.int32)
    pids = pos_ids.astype(jnp.int32)
    nids = neg_ids.reshape(-1).astype(jnp.int32)
    tailC = center_table[FULL * 128:].reshape(32, DP)
    tailX = context_table[FULL * 128:].reshape(32, DP)
    ctabP, xtabP = _pair_kernel()(center_table.T, context_table.T,
                                  tailC, tailX)
    scores = _fused_kernel()(cids, pids, nids, ctabP, xtabP)
    total = _loss(scores.reshape(B, SLOTS))
    return total[0, 0] / B


# speedup vs baseline: 7.1595x; 1.0694x over previous
import jax
import jax.numpy as jnp
from jax import lax
from jax.experimental import pallas as pl
from jax.experimental.pallas import tpu as pltpu
from jax.experimental.pallas import tpu_sc as plsc

V = 1000000
D = 64
DP = 128
VP = V // 2
B = 16384
K = 20
SLOTS = 32
NC, NS = 2, 16
NW = NC * NS
BPW = B // NW
CHB = 16
NCHK = BPW // CHB
CN = CHB * K
TCOLS = 7813
FULL = 7812
QCOLS = FULL // 2
ITERS = 123


def _pair_body(ctabT, xtabT, tailC, tailX, ctabP, xtabP,
               tb0, ob0, tb1, ob1, isem0, isem1, osem0, osem1):
    wid = lax.axis_index("s") * NC + lax.axis_index("c")
    l16 = lax.iota(jnp.int32, 16)

    def run_table(tabT, tabP):
        def col(i):
            return wid + i * NW

        def start_in(q, tb, isem):
            pltpu.async_copy(tabT.at[:, pl.ds(q * 256, 256)], tb, isem)

        @pl.when(col(0) < QCOLS)
        def _():
            start_in(col(0), tb0, isem0)

        @pl.when(col(1) < QCOLS)
        def _():
            start_in(col(1), tb1, isem1)

        def step(i, tb, ob, isem, osem):
            q = col(i)

            @pl.when(q < QCOLS)
            def _():
                pltpu.make_async_copy(
                    tabT.at[:, pl.ds(q * 256, 256)], tb, isem).wait()

                @pl.when(i >= 2)
                def _():
                    pltpu.make_async_copy(
                        ob, tabP.at[pl.ds(q * 128, 128)], osem).wait()

                @plsc.parallel_loop(0, 16, unroll=4)
                def _t(t):
                    perm = (l16 + t) & 15
                    jvh = perm >> 1
                    pv6 = (perm & 1) << 6
                    for a in range(4):
                        dvec = a * 16 + l16
                        cv = pv6 | dvec
                        for m in range(16):
                            w = plsc.load_gather(tb, [dvec, m * 16 + perm])
                            plsc.store_scatter(ob, [m * 8 + jvh, cv], w)

                pltpu.async_copy(ob, tabP.at[pl.ds(q * 128, 128)], osem)
                nxt = col(i + 2)

                @pl.when(nxt < QCOLS)
                def _():
                    start_in(nxt, tb, isem)

        def body(j, carry):
            step(2 * j, tb0, ob0, isem0, osem0)
            step(2 * j + 1, tb1, ob1, isem1, osem1)
            return carry

        lax.fori_loop(0, (ITERS + 1) // 2, body, 0)
        pltpu.make_async_copy(ob0, tabP.at[pl.ds(0, 128)], osem0).wait()
        pltpu.make_async_copy(ob1, tabP.at[pl.ds(0, 128)], osem1).wait()

    run_table(ctabT, ctabP)
    run_table(xtabT, xtabP)

    @pl.when(wid == 0)
    def _():
        def do_last(tail, tabP):
            pltpu.sync_copy(tail, ob0.at[pl.ds(0, 32)])
            pltpu.sync_copy(ob0.at[pl.ds(0, 32)],
                            tabP.at[pl.ds(FULL * 64, 32)])

        do_last(tailC, ctabP)
        do_last(tailX, xtabP)


_pair_cache = []


def _pair_kernel():
    if not _pair_cache:
        _pair_cache.append(pl.kernel(
            _pair_body,
            out_type=(
                jax.ShapeDtypeStruct((VP, DP), jnp.float32),
                jax.ShapeDtypeStruct((VP, DP), jnp.float32),
            ),
            mesh=plsc.VectorSubcoreMesh(
                core_axis_name="c", subcore_axis_name="s",
                num_cores=NC, num_subcores=NS),
            scratch_types=[
                pltpu.VMEM((D, 256), jnp.float32),
                pltpu.VMEM((128, 128), jnp.float32),
                pltpu.VMEM((D, 256), jnp.float32),
                pltpu.VMEM((128, 128), jnp.float32),
                pltpu.SemaphoreType.DMA,
                pltpu.SemaphoreType.DMA,
                pltpu.SemaphoreType.DMA,
                pltpu.SemaphoreType.DMA,
            ],
            compiler_params=pltpu.CompilerParams(needs_layout_passes=False),
        ))
    return _pair_cache[0]


def _fused_body(cids, pids, nids, ctab, xtab, s_out,
                vidxr0, pidxr0, nidxr0, vidx0, pidx0, nidx0,
                vrows0, prows0, nrows0, stage0, sem0,
                vidxr1, pidxr1, nidxr1, vidx1, pidx1, nidx1,
                vrows1, prows1, nrows1, stage1, sem1):
    wid = lax.axis_index("s") * NC + lax.axis_index("c")
    lanes = lax.iota(jnp.int32, 16)
    slots = (
        (vidxr0, pidxr0, nidxr0, vidx0, pidx0, nidx0,
         vrows0, prows0, nrows0, stage0, sem0),
        (vidxr1, pidxr1, nidxr1, vidx1, pidx1, nidx1,
         vrows1, prows1, nrows1, stage1, sem1),
    )

    def issue(c, slot):
        vidxr, pidxr, nidxr, vidx, pidx, nidx, vrows, prows, nrows, _, sem = \
            slots[slot]
        b0 = wid * BPW + c * CHB
        r0 = b0 * K
        pltpu.sync_copy(cids.at[pl.ds(b0, CHB)], vidxr.at[pl.ds(0, CHB)])
        pltpu.sync_copy(pids.at[pl.ds(b0, CHB)], pidxr.at[pl.ds(0, CHB)])
        pltpu.sync_copy(nids.at[pl.ds(r0, CN)], nidxr.at[pl.ds(0, CN)])
        for g in range(CHB // 16):
            vidx[pl.ds(g * 16, 16)] = vidxr[pl.ds(g * 16, 16)] >> 1
            pidx[pl.ds(g * 16, 16)] = pidxr[pl.ds(g * 16, 16)] >> 1
        for g in range(CN // 16):
            nidx[pl.ds(g * 16, 16)] = nidxr[pl.ds(g * 16, 16)] >> 1
        pltpu.async_copy(ctab.at[vidx], vrows, sem)
        pltpu.async_copy(xtab.at[pidx], prows, sem)
        pltpu.async_copy(xtab.at[nidx], nrows, sem)

    def process(c, slot):
        vidxr, pidxr, nidxr, vidx, pidx, nidx, vrows, prows, nrows, stage, \
            sem = slots[slot]
        b0 = wid * BPW + c * CHB
        pltpu.make_async_copy(ctab.at[vidx], vrows, sem).wait()
        pltpu.make_async_copy(xtab.at[pidx], prows, sem).wait()
        pltpu.make_async_copy(xtab.at[nidx], nrows, sem).wait()

        def per_b(b, carry_b):
            vo = (vidxr[pl.ds(b, 16)][0] & 1) * D
            v0 = vrows[b, pl.ds(vo, 16)]
            v1 = vrows[b, pl.ds(vo + 16, 16)]
            v2 = vrows[b, pl.ds(vo + 32, 16)]
            v3 = vrows[b, pl.ds(vo + 48, 16)]
            acc_a = jnp.zeros((16,), jnp.float32)
            acc_b = jnp.zeros((16,), jnp.float32)
            for k in range(K):
                r = b * K + k
                no = (nidxr[pl.ds(r, 16)][0] & 1) * D
                t = (v0 * nrows[r, pl.ds(no, 16)]
                     + v1 * nrows[r, pl.ds(no + 16, 16)]
                     + v2 * nrows[r, pl.ds(no + 32, 16)]
                     + v3 * nrows[r, pl.ds(no + 48, 16)])
                s = jnp.sum(t)
                if k < 16:
                    acc_a = jnp.where(lanes == k, s, acc_a)
                else:
                    acc_b = jnp.where(lanes == (k - 16), s, acc_b)
            po = (pidxr[pl.ds(b, 16)][0] & 1) * D
            t = (v0 * prows[b, pl.ds(po, 16)]
                 + v1 * prows[b, pl.ds(po + 16, 16)]
                 + v2 * prows[b, pl.ds(po + 32, 16)]
                 + v3 * prows[b, pl.ds(po + 48, 16)])
            acc_b = jnp.where(lanes == (K - 16), jnp.sum(t), acc_b)
            stage[pl.ds(b * SLOTS, 16)] = acc_a
            stage[pl.ds(b * SLOTS + 16, 16)] = acc_b
            return carry_b

        lax.fori_loop(0, CHB, per_b, 0)
        pltpu.sync_copy(stage, s_out.at[pl.ds(b0 * SLOTS, CHB * SLOTS)])

    issue(0, 0)

    def body(i, carry):
        issue(2 * i + 1, 1)
        process(2 * i, 0)

        @pl.when(2 * i + 2 < NCHK)
        def _():
            issue(2 * i + 2, 0)

        process(2 * i + 1, 1)
        return carry

    lax.fori_loop(0, NCHK // 2, body, 0)


_fused_cache = []


def _fused_kernel():
    if not _fused_cache:
        _fused_cache.append(pl.kernel(
            _fused_body,
            out_type=jax.ShapeDtypeStruct((B * SLOTS,), jnp.float32),
            mesh=plsc.VectorSubcoreMesh(
                core_axis_name="c", subcore_axis_name="s",
                num_cores=NC, num_subcores=NS),
            scratch_types=[
                pltpu.VMEM((CHB + 16,), jnp.int32),
                pltpu.VMEM((CHB + 16,), jnp.int32),
                pltpu.VMEM((CN + 16,), jnp.int32),
                pltpu.VMEM((CHB,), jnp.int32),
                pltpu.VMEM((CHB,), jnp.int32),
                pltpu.VMEM((CN,), jnp.int32),
                pltpu.VMEM((CHB, DP), jnp.float32),
                pltpu.VMEM((CHB, DP), jnp.float32),
                pltpu.VMEM((CN, DP), jnp.float32),
                pltpu.VMEM((CHB * SLOTS,), jnp.float32),
                pltpu.SemaphoreType.DMA,
                pltpu.VMEM((CHB + 16,), jnp.int32),
                pltpu.VMEM((CHB + 16,), jnp.int32),
                pltpu.VMEM((CN + 16,), jnp.int32),
                pltpu.VMEM((CHB,), jnp.int32),
                pltpu.VMEM((CHB,), jnp.int32),
                pltpu.VMEM((CN,), jnp.int32),
                pltpu.VMEM((CHB, DP), jnp.float32),
                pltpu.VMEM((CHB, DP), jnp.float32),
                pltpu.VMEM((CN, DP), jnp.float32),
                pltpu.VMEM((CHB * SLOTS,), jnp.float32),
                pltpu.SemaphoreType.DMA,
            ],
            compiler_params=pltpu.CompilerParams(needs_layout_passes=False),
        ))
    return _fused_cache[0]


def _logsig(x):
    return jnp.minimum(x, 0.0) - jnp.log1p(jnp.exp(-jnp.abs(x)))


def _loss_body(s_ref, o_ref):
    s = s_ref[...]
    col = lax.broadcasted_iota(jnp.int32, (B, SLOTS), 1)
    neg = jnp.where(col < K, -_logsig(-s), 0.0)
    pos = jnp.where(col == K, -_logsig(s), 0.0)
    o_ref[...] = jnp.sum(neg + pos).reshape(1, 1)


_loss = pl.pallas_call(
    _loss_body,
    out_shape=jax.ShapeDtypeStruct((1, 1), jnp.float32),
)


def kernel(center_ids, pos_ids, neg_ids, center_table, context_table):
    cids = center_ids.astype(jnp.int32)
    pids = pos_ids.astype(jnp.int32)
    nids = neg_ids.reshape(-1).astype(jnp.int32)
    tailC = center_table[FULL * 128:].reshape(32, DP)
    tailX = context_table[FULL * 128:].reshape(32, DP)
    ctabP, xtabP = _pair_kernel()(center_table.T, context_table.T,
                                  tailC, tailX)
    scores = _fused_kernel()(cids, pids, nids, ctabP, xtabP)
    total = _loss(scores.reshape(B, SLOTS))
    return total[0, 0] / B
